# baseline (speedup 1.0000x reference)
import functools

import jax
import jax.numpy as jnp
from jax import lax
from jax.experimental import pallas as pl
from jax.experimental.pallas import tpu as pltpu
from jax.experimental.pallas import tpu_sc as plsc

HE_RATIO_ = 0.005
NP_RATIO_ = 3
THR_ = 0.01

NPIX_ = 8 * 4 * 512 * 512
NW_ = 32
PER_W_ = NPIX_ // NW_
CHUNK_ = 8192
NCHUNK_ = PER_W_ // CHUNK_
L_ = 16

P1_BINS_ = 1024
P2_BINS_ = 1024
P1_SHIFT_ = 21
P2_SHIFT_ = 11
NREP_ = 2
REP_OFF_ = P1_BINS_ * L_

_mesh = plsc.VectorSubcoreMesh(core_axis_name="c", subcore_axis_name="s")


def _worker_id():
    return lax.axis_index("s") * 2 + lax.axis_index("c")


def _zero_hist(ref, nbins):
    z = jnp.zeros((L_,), jnp.float32)

    @plsc.parallel_loop(0, nbins)
    def _(j):
        ref[pl.ds(j * L_, L_)] = z


def _loss_bits(xb, yb, i):
    xv = xb[pl.ds(i * L_, L_)]
    yv = yb[pl.ds(i * L_, L_)]
    d = xv - yv
    l = d * d
    m = l >= THR_
    u = jnp.where(m, plsc.bitcast(l, jnp.int32), 0)
    v = jnp.where(m, l, 0.0)
    return u, v


def _fold_lanes(hist, hm, nbins, lane, nrep):
    lane16 = lax.shift_left(lane, 4)

    @plsc.parallel_loop(0, nbins // L_)
    def _(g):
        base = g * (L_ * L_)
        acc = plsc.load_gather(hist, [base + lane16])
        for r in range(nrep):
            for j in range(L_):
                if r == 0 and j == 0:
                    continue
                acc = acc + plsc.load_gather(
                    hist, [base + r * REP_OFF_ + lane16 + j]
                )
        hm[pl.ds(g * L_, L_)] = acc


def _stream_loop(x_hbm, y_hbm, xbufs, ybufs, sems, wid, chunk_fn):
    cps = [None, None]

    def start(c):
        b = c % 2
        base = wid * PER_W_ + c * CHUNK_
        cps[b] = (
            pltpu.async_copy(x_hbm.at[pl.ds(base, CHUNK_)], xbufs[b], sems[b]),
            pltpu.async_copy(y_hbm.at[pl.ds(base, CHUNK_)], ybufs[b], sems[2 + b]),
        )

    start(0)
    start(1)
    for c in range(NCHUNK_):
        b = c % 2
        cps[b][0].wait()
        cps[b][1].wait()
        chunk_fn(xbufs[b], ybufs[b])
        if c + 2 < NCHUNK_:
            start(c + 2)


@functools.partial(
    pl.kernel,
    out_type=jax.ShapeDtypeStruct((NW_, 2, P1_BINS_), jnp.float32),
    mesh=_mesh,
    scratch_types=[
        pltpu.VMEM((CHUNK_,), jnp.float32),
        pltpu.VMEM((CHUNK_,), jnp.float32),
        pltpu.VMEM((CHUNK_,), jnp.float32),
        pltpu.VMEM((CHUNK_,), jnp.float32),
        pltpu.VMEM((NREP_ * P1_BINS_ * L_,), jnp.float32),
        pltpu.VMEM((NREP_ * P1_BINS_ * L_,), jnp.float32),
        pltpu.VMEM((P1_BINS_,), jnp.float32),
        pltpu.VMEM((P1_BINS_,), jnp.float32),
        pltpu.SemaphoreType.DMA,
        pltpu.SemaphoreType.DMA,
        pltpu.SemaphoreType.DMA,
        pltpu.SemaphoreType.DMA,
    ],
    compiler_params=pltpu.CompilerParams(needs_layout_passes=False),
)
def _pass1(x_hbm, y_hbm, out_hbm, xb0, xb1, yb0, yb1, hcnt, hsum, hmc, hms, s0, s1, s2, s3):
    wid = _worker_id()
    _zero_hist(hcnt, NREP_ * P1_BINS_)
    _zero_hist(hsum, NREP_ * P1_BINS_)
    lane = lax.iota(jnp.int32, L_)
    ones = jnp.ones((L_,), jnp.float32)

    def chunk(xr, yr):
        @plsc.parallel_loop(0, CHUNK_ // L_, unroll=8)
        def _(i):
            u, v = _loss_bits(xr, yr, i)
            d1 = lax.shift_right_logical(u, P1_SHIFT_)
            rep = jnp.bitwise_and(i, NREP_ - 1) * REP_OFF_
            idx = rep + lax.shift_left(d1, 4) + lane
            plsc.addupdate_scatter(hcnt, [idx], ones)
            plsc.addupdate_scatter(hsum, [idx], v)

    _stream_loop(x_hbm, y_hbm, (xb0, xb1), (yb0, yb1), (s0, s1, s2, s3), wid, chunk)
    _fold_lanes(hcnt, hmc, P1_BINS_, lane, NREP_)
    _fold_lanes(hsum, hms, P1_BINS_, lane, NREP_)
    pltpu.sync_copy(hmc, out_hbm.at[wid, 0])
    pltpu.sync_copy(hms, out_hbm.at[wid, 1])


@functools.partial(
    pl.kernel,
    out_type=jax.ShapeDtypeStruct((NW_, 2, P2_BINS_), jnp.float32),
    mesh=_mesh,
    scratch_types=[
        pltpu.VMEM((CHUNK_,), jnp.float32),
        pltpu.VMEM((CHUNK_,), jnp.float32),
        pltpu.VMEM((CHUNK_,), jnp.float32),
        pltpu.VMEM((CHUNK_,), jnp.float32),
        pltpu.VMEM((P2_BINS_ * L_,), jnp.float32),
        pltpu.VMEM((P2_BINS_ * L_,), jnp.float32),
        pltpu.VMEM((P2_BINS_,), jnp.float32),
        pltpu.VMEM((P2_BINS_,), jnp.float32),
        pltpu.VMEM((L_,), jnp.int32),
        pltpu.SemaphoreType.DMA,
        pltpu.SemaphoreType.DMA,
        pltpu.SemaphoreType.DMA,
        pltpu.SemaphoreType.DMA,
    ],
    compiler_params=pltpu.CompilerParams(needs_layout_passes=False),
)
def _pass2(x_hbm, y_hbm, b1_hbm, out_hbm, xb0, xb1, yb0, yb1, hcnt, hsum, hmc, hms, b1b, s0, s1, s2, s3):
    wid = _worker_id()
    _zero_hist(hcnt, P2_BINS_)
    _zero_hist(hsum, P2_BINS_)
    pltpu.sync_copy(b1_hbm, b1b)
    b1v = b1b[...]
    lane = lax.iota(jnp.int32, L_)
    ones = jnp.ones((L_,), jnp.float32)

    def chunk(xr, yr):
        @plsc.parallel_loop(0, CHUNK_ // L_, unroll=8)
        def _(i):
            u, v = _loss_bits(xr, yr, i)
            d1 = lax.shift_right_logical(u, P1_SHIFT_)
            m2 = d1 == b1v
            d2 = jnp.bitwise_and(
                lax.shift_right_logical(u, P2_SHIFT_), P2_BINS_ - 1
            )
            idx = lax.shift_left(d2, 4) + lane
            plsc.addupdate_scatter(hcnt, [idx], ones, mask=m2)
            plsc.addupdate_scatter(hsum, [idx], v, mask=m2)

    _stream_loop(x_hbm, y_hbm, (xb0, xb1), (yb0, yb1), (s0, s1, s2, s3), wid, chunk)
    _fold_lanes(hcnt, hmc, P2_BINS_, lane, 1)
    _fold_lanes(hsum, hms, P2_BINS_, lane, 1)
    pltpu.sync_copy(hmc, out_hbm.at[wid, 0])
    pltpu.sync_copy(hms, out_hbm.at[wid, 1])


def _merge(raw, nbins):
    del nbins
    h = raw.sum(axis=0)
    return h[0], h[1]


def _locate(cnt, vsum, k):
    ac = jnp.cumsum(cnt[::-1])[::-1] - cnt
    asum = jnp.cumsum(vsum[::-1])[::-1] - vsum
    kf = k.astype(jnp.float32)
    sel = (ac < kf) & (kf <= ac + cnt)
    b = jnp.argmax(sel).astype(jnp.int32)
    return b, ac[b], asum[b]


def kernel(x, y):
    xf = x.reshape(-1)
    yf = y.reshape(-1)

    raw1 = _pass1(xf, yf)
    cnt1, sum1 = _merge(raw1, P1_BINS_)

    npos = cnt1[0].astype(jnp.int32)
    nneg = jnp.int32(NPIX_) - npos
    base = jnp.int32(int(HE_RATIO_ * NPIX_))
    k = jnp.maximum(base, jnp.minimum(jnp.int32(NP_RATIO_) * npos, nneg))

    b1, ca1, sa1 = _locate(cnt1, sum1, k)

    raw2 = _pass2(xf, yf, jnp.full((L_,), b1, jnp.int32))
    cnt2, sum2 = _merge(raw2, P2_BINS_)
    b2, ca2, sa2 = _locate(cnt2, sum2, k.astype(jnp.float32) - ca1)

    t = lax.bitcast_convert_type(
        lax.shift_left(b1, P1_SHIFT_) | lax.shift_left(b2, P2_SHIFT_),
        jnp.float32,
    )
    count_above = ca1 + ca2
    sum_above = sa1 + sa2
    s = sum_above + t * (k.astype(jnp.float32) - count_above)
    return s / k


# speedup vs baseline: 1.1484x; 1.1484x over previous
import functools

import jax
import jax.numpy as jnp
from jax import lax
from jax.experimental import pallas as pl
from jax.experimental.pallas import tpu as pltpu
from jax.experimental.pallas import tpu_sc as plsc

HE_RATIO_ = 0.005
NP_RATIO_ = 3
THR_ = 0.01

NPIX_ = 8 * 4 * 512 * 512
NW_ = 32
PER_W_ = NPIX_ // NW_
CHUNK_ = 16384
NCHUNK_ = PER_W_ // CHUNK_
L_ = 16

P1_BINS_ = 1024
P2_BINS_ = 1024
P1_SHIFT_ = 21
P2_SHIFT_ = 11

_mesh = plsc.VectorSubcoreMesh(core_axis_name="c", subcore_axis_name="s")


def _worker_id():
    return lax.axis_index("s") * 2 + lax.axis_index("c")


def _zero_hist(ref, nbins):
    z = jnp.zeros((L_,), jnp.float32)

    @plsc.parallel_loop(0, nbins)
    def _(j):
        ref[pl.ds(j * L_, L_)] = z


def _loss_bits(xb, yb, i):
    xv = xb[pl.ds(i * L_, L_)]
    yv = yb[pl.ds(i * L_, L_)]
    d = xv - yv
    l = d * d
    m = l >= THR_
    u = jnp.where(m, plsc.bitcast(l, jnp.int32), 0)
    v = jnp.where(m, l, 0.0)
    return u, v


def _fold_lanes(hist, hm, nbins, lane):
    lane16 = lax.shift_left(lane, 4)

    @plsc.parallel_loop(0, nbins // L_)
    def _(g):
        base = g * (L_ * L_)
        acc = plsc.load_gather(hist, [base + lane16])
        for j in range(1, L_):
            acc = acc + plsc.load_gather(hist, [base + lane16 + j])
        hm[pl.ds(g * L_, L_)] = acc


def _stream_loop(x_hbm, y_hbm, xbufs, ybufs, sems, wid, chunk_fn):
    cps = [None, None]

    def start(c):
        b = c % 2
        base = wid * PER_W_ + c * CHUNK_
        cps[b] = (
            pltpu.async_copy(x_hbm.at[pl.ds(base, CHUNK_)], xbufs[b], sems[b]),
            pltpu.async_copy(y_hbm.at[pl.ds(base, CHUNK_)], ybufs[b], sems[2 + b]),
        )

    start(0)
    start(1)
    for c in range(NCHUNK_):
        b = c % 2
        cps[b][0].wait()
        cps[b][1].wait()
        chunk_fn(xbufs[b], ybufs[b])
        if c + 2 < NCHUNK_:
            start(c + 2)


@functools.partial(
    pl.kernel,
    out_type=jax.ShapeDtypeStruct((NW_, 2, P1_BINS_), jnp.float32),
    mesh=_mesh,
    scratch_types=[
        pltpu.VMEM((CHUNK_,), jnp.float32),
        pltpu.VMEM((CHUNK_,), jnp.float32),
        pltpu.VMEM((CHUNK_,), jnp.float32),
        pltpu.VMEM((CHUNK_,), jnp.float32),
        pltpu.VMEM((P1_BINS_ * L_,), jnp.float32),
        pltpu.VMEM((P1_BINS_,), jnp.float32),
        pltpu.SemaphoreType.DMA,
        pltpu.SemaphoreType.DMA,
        pltpu.SemaphoreType.DMA,
        pltpu.SemaphoreType.DMA,
    ],
    compiler_params=pltpu.CompilerParams(needs_layout_passes=False),
)
def _pass1(x_hbm, y_hbm, out_hbm, xb0, xb1, yb0, yb1, hcnt, hmc, s0, s1, s2, s3):
    wid = _worker_id()
    _zero_hist(hcnt, P1_BINS_)
    lane = lax.iota(jnp.int32, L_)
    ones = jnp.ones((L_,), jnp.float32)

    def chunk(xr, yr):
        @plsc.parallel_loop(0, CHUNK_ // L_, unroll=8)
        def _(i):
            u, _v = _loss_bits(xr, yr, i)
            d1 = lax.shift_right_logical(u, P1_SHIFT_)
            idx = lax.shift_left(d1, 4) + lane
            plsc.addupdate_scatter(hcnt, [idx], ones)

    _stream_loop(x_hbm, y_hbm, (xb0, xb1), (yb0, yb1), (s0, s1, s2, s3), wid, chunk)
    _fold_lanes(hcnt, hmc, P1_BINS_, lane)
    pltpu.sync_copy(hmc, out_hbm.at[wid, 0])


@functools.partial(
    pl.kernel,
    out_type=jax.ShapeDtypeStruct((NW_, 4, P2_BINS_), jnp.float32),
    mesh=_mesh,
    scratch_types=[
        pltpu.VMEM((CHUNK_,), jnp.float32),
        pltpu.VMEM((CHUNK_,), jnp.float32),
        pltpu.VMEM((CHUNK_,), jnp.float32),
        pltpu.VMEM((CHUNK_,), jnp.float32),
        pltpu.VMEM((P2_BINS_ * L_,), jnp.float32),
        pltpu.VMEM((P2_BINS_ * L_,), jnp.float32),
        pltpu.VMEM((P2_BINS_,), jnp.float32),
        pltpu.VMEM((P2_BINS_,), jnp.float32),
        pltpu.VMEM((P2_BINS_,), jnp.float32),
        pltpu.VMEM((L_,), jnp.int32),
        pltpu.SemaphoreType.DMA,
        pltpu.SemaphoreType.DMA,
        pltpu.SemaphoreType.DMA,
        pltpu.SemaphoreType.DMA,
    ],
    compiler_params=pltpu.CompilerParams(needs_layout_passes=False),
)
def _pass2(x_hbm, y_hbm, b1_hbm, out_hbm, xb0, xb1, yb0, yb1, hcnt, hsum, hmc, hms, hma, b1b, s0, s1, s2, s3):
    wid = _worker_id()
    _zero_hist(hcnt, P2_BINS_)
    _zero_hist(hsum, P2_BINS_)
    _zero_hist(hma, P2_BINS_)
    pltpu.sync_copy(b1_hbm, b1b)
    b1v = b1b[...]
    lane = lax.iota(jnp.int32, L_)
    ones = jnp.ones((L_,), jnp.float32)
    acc = jnp.zeros((L_,), jnp.float32)

    def chunk_with_acc(xr, yr, a0):
        @plsc.parallel_loop(0, CHUNK_ // L_, unroll=8, carry=a0)
        def body(i, a):
            u, v = _loss_bits(xr, yr, i)
            d1 = lax.shift_right_logical(u, P1_SHIFT_)
            m2 = d1 == b1v
            d2 = jnp.bitwise_and(
                lax.shift_right_logical(u, P2_SHIFT_), P2_BINS_ - 1
            )
            idx = lax.shift_left(d2, 4) + lane
            plsc.addupdate_scatter(hcnt, [idx], ones, mask=m2)
            plsc.addupdate_scatter(hsum, [idx], v, mask=m2)
            return a + jnp.where(d1 > b1v, v, 0.0)

        return body

    box = [acc]

    def chunk(xr, yr):
        box[0] = chunk_with_acc(xr, yr, box[0])

    _stream_loop(x_hbm, y_hbm, (xb0, xb1), (yb0, yb1), (s0, s1, s2, s3), wid, chunk)
    _fold_lanes(hcnt, hmc, P2_BINS_, lane)
    _fold_lanes(hsum, hms, P2_BINS_, lane)
    hma[pl.ds(0, L_)] = box[0]
    pltpu.sync_copy(hmc, out_hbm.at[wid, 0])
    pltpu.sync_copy(hms, out_hbm.at[wid, 1])
    pltpu.sync_copy(hma, out_hbm.at[wid, 2])


def _locate(cnt, vsum, k):
    ac = jnp.cumsum(cnt[::-1])[::-1] - cnt
    asum = jnp.cumsum(vsum[::-1])[::-1] - vsum
    kf = k.astype(jnp.float32)
    sel = (ac < kf) & (kf <= ac + cnt)
    b = jnp.argmax(sel).astype(jnp.int32)
    return b, ac[b], asum[b]


def kernel(x, y):
    xf = x.reshape(-1)
    yf = y.reshape(-1)

    cnt1 = _pass1(xf, yf).sum(axis=0)[0]

    npos = cnt1[0].astype(jnp.int32)
    nneg = jnp.int32(NPIX_) - npos
    base = jnp.int32(int(HE_RATIO_ * NPIX_))
    k = jnp.maximum(base, jnp.minimum(jnp.int32(NP_RATIO_) * npos, nneg))

    b1, ca1, _ = _locate(cnt1, cnt1, k)

    h2 = _pass2(xf, yf, jnp.full((L_,), b1, jnp.int32)).sum(axis=0)
    cnt2, sum2, sa1 = h2[0], h2[1], h2[2].sum()
    b2, ca2, sa2 = _locate(cnt2, sum2, k.astype(jnp.float32) - ca1)

    t = lax.bitcast_convert_type(
        lax.shift_left(b1, P1_SHIFT_) | lax.shift_left(b2, P2_SHIFT_),
        jnp.float32,
    )
    count_above = ca1 + ca2
    sum_above = sa1 + sa2
    s = sum_above + t * (k.astype(jnp.float32) - count_above)
    return s / k


# speedup vs baseline: 1.5758x; 1.3721x over previous
import functools

import jax
import jax.numpy as jnp
from jax import lax
from jax.experimental import pallas as pl
from jax.experimental.pallas import tpu as pltpu
from jax.experimental.pallas import tpu_sc as plsc

HE_RATIO_ = 0.005
NP_RATIO_ = 3
THR_ = 0.01

NPIX_ = 8 * 4 * 512 * 512
NW_ = 32
ROWS_ = 512
COLS_ = 512
CROWS_ = 32
CHUNK_ = CROWS_ * COLS_
NCHUNK_ = ROWS_ // CROWS_
L_ = 16
VPR_ = COLS_ // L_

P1_BINS_ = 1024
P2_BINS_ = 1024
P1_SHIFT_ = 21
P2_SHIFT_ = 11

_mesh = plsc.VectorSubcoreMesh(core_axis_name="c", subcore_axis_name="s")


def _worker_id():
    return lax.axis_index("s") * 2 + lax.axis_index("c")


def _zero_hist(ref, nbins):
    z = jnp.zeros((L_,), jnp.float32)

    @plsc.parallel_loop(0, nbins)
    def _(j):
        ref[pl.ds(j * L_, L_)] = z


def _loss_bits(xb, yb, i):
    row = lax.shift_right_logical(i, 5)
    col = lax.shift_left(jnp.bitwise_and(i, VPR_ - 1), 4)
    xv = xb[row, pl.ds(col, L_)]
    yv = yb[row, pl.ds(col, L_)]
    d = xv - yv
    l = d * d
    m = l >= THR_
    u = jnp.where(m, plsc.bitcast(l, jnp.int32), 0)
    v = jnp.where(m, l, 0.0)
    return u, v


def _fold_lanes(hist, hm, nbins, lane):
    lane16 = lax.shift_left(lane, 4)

    @plsc.parallel_loop(0, nbins // L_)
    def _(g):
        base = g * (L_ * L_)
        acc = plsc.load_gather(hist, [base + lane16])
        for j in range(1, L_):
            acc = acc + plsc.load_gather(hist, [base + lane16 + j])
        hm[pl.ds(g * L_, L_)] = acc


def _stream_loop(x_hbm, y_hbm, xbufs, ybufs, sems, wid, chunk_fn):
    cps = [None, None]

    def start(c):
        b = c % 2
        r0 = c * CROWS_
        cps[b] = (
            pltpu.async_copy(x_hbm.at[wid, pl.ds(r0, CROWS_)], xbufs[b], sems[b]),
            pltpu.async_copy(y_hbm.at[wid, pl.ds(r0, CROWS_)], ybufs[b], sems[2 + b]),
        )

    start(0)
    start(1)
    for c in range(NCHUNK_):
        b = c % 2
        cps[b][0].wait()
        cps[b][1].wait()
        chunk_fn(xbufs[b], ybufs[b])
        if c + 2 < NCHUNK_:
            start(c + 2)


@functools.partial(
    pl.kernel,
    out_type=jax.ShapeDtypeStruct((NW_, 2, P1_BINS_), jnp.float32),
    mesh=_mesh,
    scratch_types=[
        pltpu.VMEM((CROWS_, COLS_), jnp.float32),
        pltpu.VMEM((CROWS_, COLS_), jnp.float32),
        pltpu.VMEM((CROWS_, COLS_), jnp.float32),
        pltpu.VMEM((CROWS_, COLS_), jnp.float32),
        pltpu.VMEM((P1_BINS_ * L_,), jnp.float32),
        pltpu.VMEM((P1_BINS_,), jnp.float32),
        pltpu.SemaphoreType.DMA,
        pltpu.SemaphoreType.DMA,
        pltpu.SemaphoreType.DMA,
        pltpu.SemaphoreType.DMA,
    ],
    compiler_params=pltpu.CompilerParams(needs_layout_passes=False),
)
def _pass1(x_hbm, y_hbm, out_hbm, xb0, xb1, yb0, yb1, hcnt, hmc, s0, s1, s2, s3):
    wid = _worker_id()
    _zero_hist(hcnt, P1_BINS_)
    lane = lax.iota(jnp.int32, L_)
    ones = jnp.ones((L_,), jnp.float32)

    def chunk(xr, yr):
        @plsc.parallel_loop(0, CHUNK_ // L_, unroll=8)
        def _(i):
            u, _v = _loss_bits(xr, yr, i)
            d1 = lax.shift_right_logical(u, P1_SHIFT_)
            idx = lax.shift_left(d1, 4) + lane
            plsc.addupdate_scatter(hcnt, [idx], ones)

    _stream_loop(x_hbm, y_hbm, (xb0, xb1), (yb0, yb1), (s0, s1, s2, s3), wid, chunk)
    _fold_lanes(hcnt, hmc, P1_BINS_, lane)
    pltpu.sync_copy(hmc, out_hbm.at[wid, 0])


@functools.partial(
    pl.kernel,
    out_type=jax.ShapeDtypeStruct((NW_, 4, P2_BINS_), jnp.float32),
    mesh=_mesh,
    scratch_types=[
        pltpu.VMEM((CROWS_, COLS_), jnp.float32),
        pltpu.VMEM((CROWS_, COLS_), jnp.float32),
        pltpu.VMEM((CROWS_, COLS_), jnp.float32),
        pltpu.VMEM((CROWS_, COLS_), jnp.float32),
        pltpu.VMEM((P2_BINS_ * L_,), jnp.float32),
        pltpu.VMEM((P2_BINS_ * L_,), jnp.float32),
        pltpu.VMEM((P2_BINS_,), jnp.float32),
        pltpu.VMEM((P2_BINS_,), jnp.float32),
        pltpu.VMEM((P2_BINS_,), jnp.float32),
        pltpu.VMEM((L_,), jnp.int32),
        pltpu.SemaphoreType.DMA,
        pltpu.SemaphoreType.DMA,
        pltpu.SemaphoreType.DMA,
        pltpu.SemaphoreType.DMA,
    ],
    compiler_params=pltpu.CompilerParams(needs_layout_passes=False),
)
def _pass2(x_hbm, y_hbm, b1_hbm, out_hbm, xb0, xb1, yb0, yb1, hcnt, hsum, hmc, hms, hma, b1b, s0, s1, s2, s3):
    wid = _worker_id()
    _zero_hist(hcnt, P2_BINS_)
    _zero_hist(hsum, P2_BINS_)
    _zero_hist(hma, P2_BINS_)
    pltpu.sync_copy(b1_hbm, b1b)
    b1v = b1b[...]
    lane = lax.iota(jnp.int32, L_)
    ones = jnp.ones((L_,), jnp.float32)
    acc = jnp.zeros((L_,), jnp.float32)

    def chunk_with_acc(xr, yr, a0):
        @plsc.parallel_loop(0, CHUNK_ // L_, unroll=8, carry=a0)
        def body(i, a):
            u, v = _loss_bits(xr, yr, i)
            d1 = lax.shift_right_logical(u, P1_SHIFT_)
            m2 = d1 == b1v
            d2 = jnp.bitwise_and(
                lax.shift_right_logical(u, P2_SHIFT_), P2_BINS_ - 1
            )
            idx = lax.shift_left(d2, 4) + lane
            plsc.addupdate_scatter(hcnt, [idx], ones, mask=m2)
            plsc.addupdate_scatter(hsum, [idx], v, mask=m2)
            return a + jnp.where(d1 > b1v, v, 0.0)

        return body

    box = [acc]

    def chunk(xr, yr):
        box[0] = chunk_with_acc(xr, yr, box[0])

    _stream_loop(x_hbm, y_hbm, (xb0, xb1), (yb0, yb1), (s0, s1, s2, s3), wid, chunk)
    _fold_lanes(hcnt, hmc, P2_BINS_, lane)
    _fold_lanes(hsum, hms, P2_BINS_, lane)
    hma[pl.ds(0, L_)] = box[0]
    pltpu.sync_copy(hmc, out_hbm.at[wid, 0])
    pltpu.sync_copy(hms, out_hbm.at[wid, 1])
    pltpu.sync_copy(hma, out_hbm.at[wid, 2])


def _locate(cnt, vsum, k):
    ac = jnp.cumsum(cnt[::-1])[::-1] - cnt
    asum = jnp.cumsum(vsum[::-1])[::-1] - vsum
    kf = k.astype(jnp.float32)
    sel = (ac < kf) & (kf <= ac + cnt)
    b = jnp.argmax(sel).astype(jnp.int32)
    return b, ac[b], asum[b]


def kernel(x, y):
    xf = x.reshape(NW_, ROWS_, COLS_)
    yf = y.reshape(NW_, ROWS_, COLS_)

    cnt1 = _pass1(xf, yf).sum(axis=0)[0]

    npos = cnt1[0].astype(jnp.int32)
    nneg = jnp.int32(NPIX_) - npos
    base = jnp.int32(int(HE_RATIO_ * NPIX_))
    k = jnp.maximum(base, jnp.minimum(jnp.int32(NP_RATIO_) * npos, nneg))

    b1, ca1, _ = _locate(cnt1, cnt1, k)

    h2 = _pass2(xf, yf, jnp.full((L_,), b1, jnp.int32)).sum(axis=0)
    cnt2, sum2, sa1 = h2[0], h2[1], h2[2].sum()
    b2, ca2, sa2 = _locate(cnt2, sum2, k.astype(jnp.float32) - ca1)

    t = lax.bitcast_convert_type(
        lax.shift_left(b1, P1_SHIFT_) | lax.shift_left(b2, P2_SHIFT_),
        jnp.float32,
    )
    count_above = ca1 + ca2
    sum_above = sa1 + sa2
    s = sum_above + t * (k.astype(jnp.float32) - count_above)
    return s / k


# speedup vs baseline: 1.5819x; 1.0039x over previous
import functools

import jax
import jax.numpy as jnp
from jax import lax
from jax.experimental import pallas as pl
from jax.experimental.pallas import tpu as pltpu
from jax.experimental.pallas import tpu_sc as plsc

HE_RATIO_ = 0.005
NP_RATIO_ = 3
THR_ = 0.01

NPIX_ = 8 * 4 * 512 * 512
NW_ = 32
ROWS_ = 512
COLS_ = 512
CROWS_ = 32
CHUNK_ = CROWS_ * COLS_
NCHUNK_ = ROWS_ // CROWS_
L_ = 16
VPR_ = COLS_ // L_

P1_BINS_ = 1024
P2_BINS_ = 1024
P1_SHIFT_ = 21
P2_SHIFT_ = 11

_mesh = plsc.VectorSubcoreMesh(core_axis_name="c", subcore_axis_name="s")


def _worker_id():
    return lax.axis_index("s") * 2 + lax.axis_index("c")


def _zero_hist(ref, nbins):
    z = jnp.zeros((L_,), jnp.float32)

    @plsc.parallel_loop(0, nbins)
    def _(j):
        ref[pl.ds(j * L_, L_)] = z


def _loss_bits(xb, yb, i):
    row = lax.shift_right_logical(i, 5)
    col = lax.shift_left(jnp.bitwise_and(i, VPR_ - 1), 4)
    xv = xb[row, pl.ds(col, L_)]
    yv = yb[row, pl.ds(col, L_)]
    d = xv - yv
    l = d * d
    m = l >= THR_
    u = jnp.where(m, plsc.bitcast(l, jnp.int32), 0)
    v = jnp.where(m, l, 0.0)
    return u, v


def _fold_lanes(hist, hm, nbins, lane):
    lane16 = lax.shift_left(lane, 4)

    @plsc.parallel_loop(0, nbins // L_)
    def _(g):
        base = g * (L_ * L_)
        acc = plsc.load_gather(hist, [base + lane16])
        for j in range(1, L_):
            acc = acc + plsc.load_gather(hist, [base + lane16 + j])
        hm[pl.ds(g * L_, L_)] = acc


def _stream_loop(x_hbm, y_hbm, xbufs, ybufs, sems, wid, chunk_fn):
    cps = [None, None]

    def start(c):
        b = c % 2
        r0 = c * CROWS_
        cps[b] = (
            pltpu.async_copy(x_hbm.at[wid, pl.ds(r0, CROWS_)], xbufs[b], sems[b]),
            pltpu.async_copy(y_hbm.at[wid, pl.ds(r0, CROWS_)], ybufs[b], sems[2 + b]),
        )

    start(0)
    start(1)
    for c in range(NCHUNK_):
        b = c % 2
        cps[b][0].wait()
        cps[b][1].wait()
        chunk_fn(xbufs[b], ybufs[b])
        if c + 2 < NCHUNK_:
            start(c + 2)


@functools.partial(
    pl.kernel,
    out_type=jax.ShapeDtypeStruct((NW_, 2, P1_BINS_), jnp.float32),
    mesh=_mesh,
    scratch_types=[
        pltpu.VMEM((CROWS_, COLS_), jnp.float32),
        pltpu.VMEM((CROWS_, COLS_), jnp.float32),
        pltpu.VMEM((CROWS_, COLS_), jnp.float32),
        pltpu.VMEM((CROWS_, COLS_), jnp.float32),
        pltpu.VMEM((P1_BINS_ * L_,), jnp.float32),
        pltpu.VMEM((P1_BINS_,), jnp.float32),
        pltpu.SemaphoreType.DMA,
        pltpu.SemaphoreType.DMA,
        pltpu.SemaphoreType.DMA,
        pltpu.SemaphoreType.DMA,
    ],
    compiler_params=pltpu.CompilerParams(needs_layout_passes=False),
)
def _pass1(x_hbm, y_hbm, out_hbm, xb0, xb1, yb0, yb1, hcnt, hmc, s0, s1, s2, s3):
    wid = _worker_id()
    _zero_hist(hcnt, P1_BINS_)
    lane = lax.iota(jnp.int32, L_)
    ones = jnp.ones((L_,), jnp.float32)

    def chunk(xr, yr):
        @plsc.parallel_loop(0, CHUNK_ // L_, unroll=4)
        def _(i):
            u, _v = _loss_bits(xr, yr, i)
            d1 = lax.shift_right_logical(u, P1_SHIFT_)
            idx = lax.shift_left(d1, 4) + lane
            plsc.addupdate_scatter(hcnt, [idx], ones)

    _stream_loop(x_hbm, y_hbm, (xb0, xb1), (yb0, yb1), (s0, s1, s2, s3), wid, chunk)
    _fold_lanes(hcnt, hmc, P1_BINS_, lane)
    pltpu.sync_copy(hmc, out_hbm.at[wid, 0])


@functools.partial(
    pl.kernel,
    out_type=jax.ShapeDtypeStruct((NW_, 4, P2_BINS_), jnp.float32),
    mesh=_mesh,
    scratch_types=[
        pltpu.VMEM((CROWS_, COLS_), jnp.float32),
        pltpu.VMEM((CROWS_, COLS_), jnp.float32),
        pltpu.VMEM((CROWS_, COLS_), jnp.float32),
        pltpu.VMEM((CROWS_, COLS_), jnp.float32),
        pltpu.VMEM((P2_BINS_ * L_,), jnp.float32),
        pltpu.VMEM((P2_BINS_ * L_,), jnp.float32),
        pltpu.VMEM((P2_BINS_,), jnp.float32),
        pltpu.VMEM((P2_BINS_,), jnp.float32),
        pltpu.VMEM((P2_BINS_,), jnp.float32),
        pltpu.VMEM((L_,), jnp.int32),
        pltpu.SemaphoreType.DMA,
        pltpu.SemaphoreType.DMA,
        pltpu.SemaphoreType.DMA,
        pltpu.SemaphoreType.DMA,
    ],
    compiler_params=pltpu.CompilerParams(needs_layout_passes=False),
)
def _pass2(x_hbm, y_hbm, b1_hbm, out_hbm, xb0, xb1, yb0, yb1, hcnt, hsum, hmc, hms, hma, b1b, s0, s1, s2, s3):
    wid = _worker_id()
    _zero_hist(hcnt, P2_BINS_)
    _zero_hist(hsum, P2_BINS_)
    _zero_hist(hma, P2_BINS_)
    pltpu.sync_copy(b1_hbm, b1b)
    b1v = b1b[...]
    lane = lax.iota(jnp.int32, L_)
    ones = jnp.ones((L_,), jnp.float32)
    acc = jnp.zeros((L_,), jnp.float32)

    def chunk_with_acc(xr, yr, a0):
        @plsc.parallel_loop(0, CHUNK_ // L_, unroll=4, carry=a0)
        def body(i, a):
            u, v = _loss_bits(xr, yr, i)
            d1 = lax.shift_right_logical(u, P1_SHIFT_)
            m2 = d1 == b1v
            d2 = jnp.bitwise_and(
                lax.shift_right_logical(u, P2_SHIFT_), P2_BINS_ - 1
            )
            idx = lax.shift_left(d2, 4) + lane
            plsc.addupdate_scatter(hcnt, [idx], ones, mask=m2)
            plsc.addupdate_scatter(hsum, [idx], v, mask=m2)
            return a + jnp.where(d1 > b1v, v, 0.0)

        return body

    box = [acc]

    def chunk(xr, yr):
        box[0] = chunk_with_acc(xr, yr, box[0])

    _stream_loop(x_hbm, y_hbm, (xb0, xb1), (yb0, yb1), (s0, s1, s2, s3), wid, chunk)
    _fold_lanes(hcnt, hmc, P2_BINS_, lane)
    _fold_lanes(hsum, hms, P2_BINS_, lane)
    hma[pl.ds(0, L_)] = box[0]
    pltpu.sync_copy(hmc, out_hbm.at[wid, 0])
    pltpu.sync_copy(hms, out_hbm.at[wid, 1])
    pltpu.sync_copy(hma, out_hbm.at[wid, 2])


def _locate(cnt, vsum, k):
    ac = jnp.cumsum(cnt[::-1])[::-1] - cnt
    asum = jnp.cumsum(vsum[::-1])[::-1] - vsum
    kf = k.astype(jnp.float32)
    sel = (ac < kf) & (kf <= ac + cnt)
    b = jnp.argmax(sel).astype(jnp.int32)
    return b, ac[b], asum[b]


def kernel(x, y):
    xf = x.reshape(NW_, ROWS_, COLS_)
    yf = y.reshape(NW_, ROWS_, COLS_)

    cnt1 = _pass1(xf, yf).sum(axis=0)[0]

    npos = cnt1[0].astype(jnp.int32)
    nneg = jnp.int32(NPIX_) - npos
    base = jnp.int32(int(HE_RATIO_ * NPIX_))
    k = jnp.maximum(base, jnp.minimum(jnp.int32(NP_RATIO_) * npos, nneg))

    b1, ca1, _ = _locate(cnt1, cnt1, k)

    h2 = _pass2(xf, yf, jnp.full((L_,), b1, jnp.int32)).sum(axis=0)
    cnt2, sum2, sa1 = h2[0], h2[1], h2[2].sum()
    b2, ca2, sa2 = _locate(cnt2, sum2, k.astype(jnp.float32) - ca1)

    t = lax.bitcast_convert_type(
        lax.shift_left(b1, P1_SHIFT_) | lax.shift_left(b2, P2_SHIFT_),
        jnp.float32,
    )
    count_above = ca1 + ca2
    sum_above = sa1 + sa2
    s = sum_above + t * (k.astype(jnp.float32) - count_above)
    return s / k
